# baseline (speedup 1.0000x reference)
import functools
import math

import jax
import jax.numpy as jnp
from jax import lax
from jax.experimental import pallas as pl
from jax.experimental.pallas import tpu as pltpu
from jax.experimental.pallas import tpu_sc as plsc

_D = 64
_SCALE = math.sqrt(float(_D))
_NC, _NS = 2, 16
_NW = _NC * _NS
_CH = 128
_LANES = 16


def _scale_buf(buf, rows):
    rows_per_iter = 4

    def body(i, carry):
        for rr in range(rows_per_iter):
            r = i * rows_per_iter + rr
            for k in range(_D // _LANES):
                sl = pl.ds(k * _LANES, _LANES)
                buf[r, sl] = buf[r, sl] * _SCALE
        return carry

    lax.fori_loop(0, rows // rows_per_iter, body, 0)


@functools.partial(jax.jit, static_argnames=("nch",))
def _embed(W, idx, nch):
    @functools.partial(
        pl.kernel,
        out_type=jax.ShapeDtypeStruct((_NW, nch, _CH, _D), jnp.float32),
        mesh=plsc.VectorSubcoreMesh(core_axis_name="c", subcore_axis_name="s"),
        compiler_params=pltpu.CompilerParams(use_tc_tiling_on_sc=False),
        scratch_types=[
            pltpu.VMEM((nch, _CH), jnp.int32),
            pltpu.VMEM((_CH, _D), jnp.float32),
            pltpu.VMEM((_CH, _D), jnp.float32),
            pltpu.SemaphoreType.DMA,
            pltpu.SemaphoreType.DMA,
        ],
    )
    def emb(table, idx_h, out_h, idx_v, buf0, buf1, sem0, sem1):
        wid = lax.axis_index("s") * _NC + lax.axis_index("c")
        pltpu.sync_copy(idx_h.at[wid], idx_v)
        bufs = (buf0, buf1)
        sems = (sem0, sem1)

        pltpu.async_copy(table.at[idx_v.at[0]], buf0, sem0)

        def outer(i, carry):
            for b in range(2):
                c = i * 2 + b
                nxt = c + 1
                nb = (b + 1) % 2

                @pl.when(nxt < nch)
                def _():
                    pltpu.async_copy(table.at[idx_v.at[nxt]], bufs[nb], sems[nb])

                pltpu.make_async_copy(table.at[idx_v.at[c]], bufs[b], sems[b]).wait()
                _scale_buf(bufs[b], _CH)
                pltpu.sync_copy(bufs[b], out_h.at[wid, c])
            return carry

        lax.fori_loop(0, nch // 2, outer, 0)

    return emb(W, idx)


def kernel(x, W):
    S, T = x.shape
    B = S * T
    assert B % (_NW * _CH) == 0
    nch = B // (_NW * _CH)
    idx = x.reshape(_NW, nch, _CH)
    out = _embed(W, idx, nch)
    return out.reshape(S, T, _D)
